# baseline (speedup 1.0000x reference)
import functools

import jax
import jax.numpy as jnp
from jax import lax
from jax.experimental import pallas as pl
from jax.experimental.pallas import tpu as pltpu
from jax.experimental.pallas import tpu_sc as plsc

NN = 10000
EE = 320000
IN_C = 128
HID = 64
OUT_C = 128

NC = 2
NS = 16
NW = NC * NS

N2 = 10240
E2 = 327680
IDX_ROWS = E2 // 128
ROWS_PER_W = IDX_ROWS // NW
SUP = 5
NSUP = ROWS_PER_W // SUP
RPT = N2 // NS
CR = 128
NCR = RPT // CR


def _sc_mesh():
    return plsc.VectorSubcoreMesh(core_axis_name="c", subcore_axis_name="s",
                                  num_cores=NC, num_subcores=NS)


def _agg_phase(table, src2, dst2, acc, src_v, dst_v, rows_v, gsem, ssem,
               w, hbm_dummy, sup, dacc=None, ones_v=None, ones16=None):
    base = w * ROWS_PER_W
    nsup = ROWS_PER_W // sup

    def _scatter_waits(j):
        pltpu.make_async_copy(hbm_dummy,
                              rows_v.at[pl.ds(j * 128, 128)],
                              ssem.at[j]).wait()
        if dacc is not None:
            pltpu.make_async_copy(ones16, ones_v, ssem.at[j]).wait()

    def chunk(i, carry):
        ro = base + i * sup
        par = lax.rem(i, 2)
        pltpu.sync_copy(src2.at[pl.ds(ro, sup)], src_v.at[par])
        pltpu.sync_copy(dst2.at[pl.ds(ro, sup)], dst_v.at[par])

        @pl.when(i > 0)
        def _():
            for j in range(sup):
                _scatter_waits(j)

        gcs = [pltpu.async_copy(table.at[src_v.at[par, j]],
                                rows_v.at[pl.ds(j * 128, 128)],
                                gsem.at[j])
               for j in range(sup)]
        for j in range(sup):
            gcs[j].wait()
            pltpu.async_copy(rows_v.at[pl.ds(j * 128, 128)],
                             acc.at[dst_v.at[par, j]], ssem.at[j], add=True)
            if dacc is not None:
                pltpu.async_copy(ones_v, dacc.at[dst_v.at[par, j]],
                                 ssem.at[j], add=True)
        return carry

    lax.fori_loop(0, nsup, chunk, 0)
    for j in range(sup):
        _scatter_waits(j)


def _combine_chunk(p, dinv64, z, b1_v, relu, rows_v, cr0):
    pltpu.sync_copy(p.at[0, pl.ds(cr0, CR)], rows_v.at[pl.ds(0, CR)])
    pltpu.sync_copy(p.at[1, pl.ds(cr0, CR)], rows_v.at[pl.ds(CR, CR)])
    pltpu.sync_copy(dinv64.at[pl.ds(cr0, CR)], rows_v.at[pl.ds(2 * CR, CR)])
    if z is not None:
        pltpu.sync_copy(z.at[pl.ds(cr0, CR)], rows_v.at[pl.ds(3 * CR, CR)])

    def rowpair(ri, carry):
        for rr in range(2):
            r = ri * 2 + rr
            for j in range(HID // 16):
                sl = pl.ds(j * 16, 16)
                t = (rows_v[r, sl] + rows_v[CR + r, sl]) * rows_v[2 * CR + r, sl]
                if z is not None:
                    t = t + rows_v[3 * CR + r, sl]
                if relu:
                    t = jnp.maximum(t + b1_v[sl], 0.0)
                rows_v[4 * CR + r, sl] = t
        return carry

    lax.fori_loop(0, CR // 2, rowpair, 0)


def _make_sc_app(first, z_add, relu, table_out):
    out_type = [jax.ShapeDtypeStruct((NC, N2, HID), jnp.float32)]
    if first:
        out_type.append(jax.ShapeDtypeStruct((NC, N2, 16), jnp.float32))
    if table_out:
        out_type.append(jax.ShapeDtypeStruct((N2, HID), jnp.float32))

    sup = 4 if first else SUP
    scratch = [
        pltpu.VMEM_SHARED((N2, HID), jnp.float32),
        pltpu.VMEM_SHARED((N2, HID), jnp.float32),
        pltpu.VMEM((2, sup, 128), jnp.int32),
        pltpu.VMEM((2, sup, 128), jnp.int32),
        pltpu.VMEM((sup * 128, HID), jnp.float32),
        pltpu.SemaphoreType.DMA((sup,)),
        pltpu.SemaphoreType.DMA((sup,)),
    ]
    if first:
        scratch += [pltpu.VMEM_SHARED((N2, 16), jnp.float32),
                    pltpu.VMEM((128, 16), jnp.float32)]
    else:
        scratch += [pltpu.VMEM((HID,), jnp.float32)]

    def body(*refs):
        if first:
            (tblin, src2, dst2, zeros64, zeros16, ones16, pout, dout,
             table, acc, src_v, dst_v, rows_v, gsem, ssem,
             dacc, ones_v) = refs
        else:
            p_in, src2, dst2, zeros64, dinv64 = refs[:5]
            i = 5
            z = b1 = None
            if z_add:
                z = refs[i]
                i += 1
            if relu:
                b1 = refs[i]
                i += 1
            pout = refs[i]
            i += 1
            tout = None
            if table_out:
                tout = refs[i]
                i += 1
            (table, acc, src_v, dst_v, rows_v, gsem, ssem,
             b1_v) = refs[i:]

        c = lax.axis_index("c")
        s = lax.axis_index("s")
        w = s * NC + c
        r0 = s * RPT

        if first:
            pltpu.sync_copy(tblin.at[pl.ds(r0, RPT)], table.at[pl.ds(r0, RPT)])
            pltpu.sync_copy(zeros16.at[pl.ds(r0, RPT)],
                            dacc.at[pl.ds(r0, RPT)])
            pltpu.sync_copy(ones16, ones_v)
            hbm_dummy = tblin.at[pl.ds(0, 128)]
        else:
            if relu:
                pltpu.sync_copy(b1, b1_v)
            for k in range(NCR):
                cr0 = r0 + k * CR
                _combine_chunk(p_in, dinv64, z, b1_v, relu, rows_v, cr0)
                pltpu.sync_copy(rows_v.at[pl.ds(4 * CR, CR)],
                                table.at[pl.ds(cr0, CR)])
                if table_out:
                    @pl.when(c == 0)
                    def _():
                        pltpu.sync_copy(rows_v.at[pl.ds(4 * CR, CR)],
                                        tout.at[pl.ds(cr0, CR)])
            hbm_dummy = p_in.at[0, pl.ds(0, 128)]

        pltpu.sync_copy(zeros64.at[pl.ds(r0, RPT)], acc.at[pl.ds(r0, RPT)])
        plsc.subcore_barrier()

        if first:
            _agg_phase(table, src2, dst2, acc, src_v, dst_v, rows_v, gsem,
                       ssem, w, hbm_dummy, sup, dacc=dacc, ones_v=ones_v,
                       ones16=ones16)
        else:
            _agg_phase(table, src2, dst2, acc, src_v, dst_v, rows_v, gsem,
                       ssem, w, hbm_dummy, sup)
        plsc.subcore_barrier()

        pltpu.sync_copy(acc.at[pl.ds(r0, RPT)], pout.at[c, pl.ds(r0, RPT)])
        if first:
            pltpu.sync_copy(dacc.at[pl.ds(r0, RPT)],
                            dout.at[c, pl.ds(r0, RPT)])

    return pl.kernel(
        body,
        out_type=tuple(out_type) if len(out_type) > 1 else out_type[0],
        mesh=_sc_mesh(),
        compiler_params=pltpu.CompilerParams(use_tc_tiling_on_sc=False),
        scratch_types=tuple(scratch),
    )


_app_first = _make_sc_app(first=True, z_add=False, relu=False, table_out=False)
_app_z = _make_sc_app(first=False, z_add=True, relu=False, table_out=False)
_app_relu = _make_sc_app(first=False, z_add=True, relu=True, table_out=True)
_app_plain = _make_sc_app(first=False, z_add=False, relu=False, table_out=True)


def _zmm3(x2, W1):
    def body(x_ref, w_ref, z_ref):
        z_ref[...] = jnp.dot(x_ref[...], w_ref[3],
                             preferred_element_type=jnp.float32)

    return pl.pallas_call(
        body,
        out_shape=jax.ShapeDtypeStruct((N2, HID), jnp.float32),
    )(x2, W1)


def _zmm012(x2, W1):
    def body(x_ref, w_ref, z_ref):
        for k in range(3):
            z_ref[k] = jnp.dot(x_ref[...], w_ref[k],
                               preferred_element_type=jnp.float32)

    return pl.pallas_call(
        body,
        out_shape=jax.ShapeDtypeStruct((3, N2, HID), jnp.float32),
    )(x2, W1)


def _dinv(degp):
    def body(dp_ref, o_ref):
        dv = 1.0 / jnp.maximum(dp_ref[0, :, :1] + dp_ref[1, :, :1], 1.0)
        o_ref[...] = jnp.broadcast_to(dv, (N2, HID))

    return pl.pallas_call(
        body, out_shape=jax.ShapeDtypeStruct((N2, HID), jnp.float32),
    )(degp)


def _fin(h, a1, a2, p, dinv64, W2, b2):
    def body(h_ref, a1_ref, a2_ref, p_ref, di_ref, w_ref, b_ref, y_ref):
        a3 = (p_ref[0] + p_ref[1]) * di_ref[...]
        acc = jnp.dot(h_ref[...], w_ref[0], preferred_element_type=jnp.float32)
        acc += jnp.dot(a1_ref[...], w_ref[1], preferred_element_type=jnp.float32)
        acc += jnp.dot(a2_ref[...], w_ref[2], preferred_element_type=jnp.float32)
        acc += jnp.dot(a3, w_ref[3], preferred_element_type=jnp.float32)
        y_ref[...] = acc + b_ref[...]

    return pl.pallas_call(
        body, out_shape=jax.ShapeDtypeStruct((N2, OUT_C), jnp.float32),
    )(h, a1, a2, p, dinv64, W2, b2)


def kernel(x, edge_index, W1, b1, W2, b2):
    src = edge_index[0]
    dst = edge_index[1]
    pad = E2 - EE
    pidx = lax.iota(jnp.int32, pad)
    src2 = jnp.concatenate([src, pidx % 128]).reshape(IDX_ROWS, 128)
    dst2 = jnp.concatenate([dst, NN + (pidx % 8)]).reshape(IDX_ROWS, 128)
    x2 = jnp.pad(x, ((0, N2 - NN), (0, 0)))
    zeros64 = jnp.zeros((N2, HID), jnp.float32)
    zeros16 = jnp.zeros((N2, 16), jnp.float32)
    ones16 = jnp.ones((128, 16), jnp.float32)

    z3 = _zmm3(x2, W1)
    p, degp = _app_first(z3, src2, dst2, zeros64, zeros16, ones16)
    z = _zmm012(x2, W1)
    dinv64 = _dinv(degp)
    p = _app_z(p, src2, dst2, zeros64, dinv64, z[2])
    p = _app_z(p, src2, dst2, zeros64, dinv64, z[1])
    p, h = _app_relu(p, src2, dst2, zeros64, dinv64, z[0], b1)
    p, a1 = _app_plain(p, src2, dst2, zeros64, dinv64)
    p, a2 = _app_plain(p, src2, dst2, zeros64, dinv64)
    y2 = _fin(h, a1, a2, p, dinv64, W2, b2)
    return y2[:NN]


# speedup vs baseline: 1.0812x; 1.0812x over previous
import functools

import jax
import jax.numpy as jnp
from jax import lax
from jax.experimental import pallas as pl
from jax.experimental.pallas import tpu as pltpu
from jax.experimental.pallas import tpu_sc as plsc

NN = 10000
EE = 320000
IN_C = 128
HID = 64
OUT_C = 128

NC = 2
NS = 16
NW = NC * NS

N2 = 10240
E2 = 327680
IDX_ROWS = E2 // 128
ROWS_PER_W = IDX_ROWS // NW
SUP = 5
NSUP = ROWS_PER_W // SUP
RPT = N2 // NS
CR = 64
NCR = RPT // CR


def _sc_mesh():
    return plsc.VectorSubcoreMesh(core_axis_name="c", subcore_axis_name="s",
                                  num_cores=NC, num_subcores=NS)


def _agg_phase(table, src2, dst2, acc, src_v, dst_v, rows_v, gsem, ssem,
               w, hbm_dummy, sup, dacc=None, ones_v=None, ones16=None):
    base = w * ROWS_PER_W
    nsup = ROWS_PER_W // sup

    def _scatter_waits(j):
        pltpu.make_async_copy(hbm_dummy,
                              rows_v.at[pl.ds(j * 128, 128)],
                              ssem.at[j]).wait()
        if dacc is not None:
            pltpu.make_async_copy(ones16, ones_v, ssem.at[j]).wait()

    def chunk(i, carry):
        ro = base + i * sup
        par = lax.rem(i, 2)
        pltpu.sync_copy(src2.at[pl.ds(ro, sup)], src_v.at[par])
        pltpu.sync_copy(dst2.at[pl.ds(ro, sup)], dst_v.at[par])

        @pl.when(i > 0)
        def _():
            for j in range(sup):
                _scatter_waits(j)

        gcs = [pltpu.async_copy(table.at[src_v.at[par, j]],
                                rows_v.at[pl.ds(j * 128, 128)],
                                gsem.at[j])
               for j in range(sup)]
        for j in range(sup):
            gcs[j].wait()
            pltpu.async_copy(rows_v.at[pl.ds(j * 128, 128)],
                             acc.at[dst_v.at[par, j]], ssem.at[j], add=True)
            if dacc is not None:
                pltpu.async_copy(ones_v, dacc.at[dst_v.at[par, j]],
                                 ssem.at[j], add=True)
        return carry

    lax.fori_loop(0, nsup, chunk, 0)
    for j in range(sup):
        _scatter_waits(j)


def _combine_phase(p, dinv64, z, b1_v, relu, rows_v, table, tout, c, r0,
                   csem, osem):
    nin = 4 if z is not None else 3

    def prefetch(k, par):
        base = par * 4 * CR
        cr0 = r0 + k * CR
        pltpu.async_copy(p.at[0, pl.ds(cr0, CR)],
                         rows_v.at[pl.ds(base, CR)], csem.at[par])
        pltpu.async_copy(p.at[1, pl.ds(cr0, CR)],
                         rows_v.at[pl.ds(base + CR, CR)], csem.at[par])
        pltpu.async_copy(dinv64.at[pl.ds(cr0, CR)],
                         rows_v.at[pl.ds(base + 2 * CR, CR)], csem.at[par])
        if z is not None:
            pltpu.async_copy(z.at[pl.ds(cr0, CR)],
                             rows_v.at[pl.ds(base + 3 * CR, CR)],
                             csem.at[par])

    def wait_in(par):
        base = par * 4 * CR
        for _ in range(nin):
            pltpu.make_async_copy(p.at[0, pl.ds(0, CR)],
                                  rows_v.at[pl.ds(base, CR)],
                                  csem.at[par]).wait()

    def compute_and_store(k, par):
        base = par * 4 * CR
        ob = 8 * CR + par * CR
        cr0 = r0 + k * CR

        def rowpair(ri, carry2):
            for rr in range(2):
                r = ri * 2 + rr
                for j in range(HID // 16):
                    sl = pl.ds(j * 16, 16)
                    t = ((rows_v[base + r, sl] + rows_v[base + CR + r, sl])
                         * rows_v[base + 2 * CR + r, sl])
                    if z is not None:
                        t = t + rows_v[base + 3 * CR + r, sl]
                    if relu:
                        t = jnp.maximum(t + b1_v[sl], 0.0)
                    rows_v[ob + r, sl] = t
            return carry2

        lax.fori_loop(0, CR // 2, rowpair, 0)
        pltpu.sync_copy(rows_v.at[pl.ds(ob, CR)],
                        table.at[pl.ds(cr0, CR)])
        if tout is not None:
            @pl.when(c == 0)
            def _():
                pltpu.sync_copy(rows_v.at[pl.ds(ob, CR)],
                                tout.at[pl.ds(cr0, CR)])

    prefetch(0, 0)

    def pairbody(kk, carry):
        k0 = kk * 2
        wait_in(0)
        prefetch(k0 + 1, 1)
        compute_and_store(k0, 0)
        wait_in(1)

        @pl.when(kk + 1 < NCR // 2)
        def _():
            prefetch(k0 + 2, 0)
        compute_and_store(k0 + 1, 1)
        return carry

    lax.fori_loop(0, NCR // 2, pairbody, 0)


def _make_sc_app(first, z_add, relu, table_out):
    out_type = [jax.ShapeDtypeStruct((NC, N2, HID), jnp.float32)]
    if first:
        out_type.append(jax.ShapeDtypeStruct((NC, N2, 16), jnp.float32))
    if table_out:
        out_type.append(jax.ShapeDtypeStruct((N2, HID), jnp.float32))

    sup = 4 if first else SUP
    scratch = [
        pltpu.VMEM_SHARED((N2, HID), jnp.float32),
        pltpu.VMEM_SHARED((N2, HID), jnp.float32),
        pltpu.VMEM((2, sup, 128), jnp.int32),
        pltpu.VMEM((2, sup, 128), jnp.int32),
        pltpu.VMEM((sup * 128, HID), jnp.float32),
        pltpu.SemaphoreType.DMA((sup,)),
        pltpu.SemaphoreType.DMA((sup,)),
    ]
    if first:
        scratch += [pltpu.VMEM_SHARED((N2, 16), jnp.float32),
                    pltpu.VMEM((128, 16), jnp.float32)]
    else:
        scratch += [pltpu.VMEM((HID,), jnp.float32),
                    pltpu.SemaphoreType.DMA((2,)),
                    pltpu.SemaphoreType.DMA((2,))]

    def body(*refs):
        if first:
            (tblin, src2, dst2, zeros64, zeros16, ones16, pout, dout,
             table, acc, src_v, dst_v, rows_v, gsem, ssem,
             dacc, ones_v) = refs
        else:
            p_in, src2, dst2, zeros64, dinv64 = refs[:5]
            i = 5
            z = b1 = None
            if z_add:
                z = refs[i]
                i += 1
            if relu:
                b1 = refs[i]
                i += 1
            pout = refs[i]
            i += 1
            tout = None
            if table_out:
                tout = refs[i]
                i += 1
            (table, acc, src_v, dst_v, rows_v, gsem, ssem,
             b1_v, csem, osem) = refs[i:]

        c = lax.axis_index("c")
        s = lax.axis_index("s")
        w = s * NC + c
        r0 = s * RPT

        if first:
            pltpu.sync_copy(tblin.at[pl.ds(r0, RPT)], table.at[pl.ds(r0, RPT)])
            pltpu.sync_copy(zeros16.at[pl.ds(r0, RPT)],
                            dacc.at[pl.ds(r0, RPT)])
            pltpu.sync_copy(ones16, ones_v)
            hbm_dummy = tblin.at[pl.ds(0, 128)]
        else:
            if relu:
                pltpu.sync_copy(b1, b1_v)
            _combine_phase(p_in, dinv64, z, b1_v, relu, rows_v, table, tout,
                           c, r0, csem, osem)
            hbm_dummy = p_in.at[0, pl.ds(0, 128)]

        pltpu.sync_copy(zeros64.at[pl.ds(r0, RPT)], acc.at[pl.ds(r0, RPT)])
        plsc.subcore_barrier()

        if first:
            _agg_phase(table, src2, dst2, acc, src_v, dst_v, rows_v, gsem,
                       ssem, w, hbm_dummy, sup, dacc=dacc, ones_v=ones_v,
                       ones16=ones16)
        else:
            _agg_phase(table, src2, dst2, acc, src_v, dst_v, rows_v, gsem,
                       ssem, w, hbm_dummy, sup)
        plsc.subcore_barrier()

        pltpu.sync_copy(acc.at[pl.ds(r0, RPT)], pout.at[c, pl.ds(r0, RPT)])
        if first:
            pltpu.sync_copy(dacc.at[pl.ds(r0, RPT)],
                            dout.at[c, pl.ds(r0, RPT)])

    return pl.kernel(
        body,
        out_type=tuple(out_type) if len(out_type) > 1 else out_type[0],
        mesh=_sc_mesh(),
        compiler_params=pltpu.CompilerParams(use_tc_tiling_on_sc=False),
        scratch_types=tuple(scratch),
    )


_app_first = _make_sc_app(first=True, z_add=False, relu=False, table_out=False)
_app_z = _make_sc_app(first=False, z_add=True, relu=False, table_out=False)
_app_relu = _make_sc_app(first=False, z_add=True, relu=True, table_out=True)
_app_plain = _make_sc_app(first=False, z_add=False, relu=False, table_out=True)


def _zmm3(x2, W1):
    def body(x_ref, w_ref, z_ref):
        z_ref[...] = jnp.dot(x_ref[...], w_ref[3],
                             preferred_element_type=jnp.float32)

    return pl.pallas_call(
        body,
        out_shape=jax.ShapeDtypeStruct((N2, HID), jnp.float32),
    )(x2, W1)


def _zmm012(x2, W1):
    def body(x_ref, w_ref, z_ref):
        for k in range(3):
            z_ref[k] = jnp.dot(x_ref[...], w_ref[k],
                               preferred_element_type=jnp.float32)

    return pl.pallas_call(
        body,
        out_shape=jax.ShapeDtypeStruct((3, N2, HID), jnp.float32),
    )(x2, W1)


def _dinv(degp):
    def body(dp_ref, o_ref):
        dv = 1.0 / jnp.maximum(dp_ref[0, :, :1] + dp_ref[1, :, :1], 1.0)
        o_ref[...] = jnp.broadcast_to(dv, (N2, HID))

    return pl.pallas_call(
        body, out_shape=jax.ShapeDtypeStruct((N2, HID), jnp.float32),
    )(degp)


def _fin(h, a1, a2, p, dinv64, W2, b2):
    def body(h_ref, a1_ref, a2_ref, p_ref, di_ref, w_ref, b_ref, y_ref):
        a3 = (p_ref[0] + p_ref[1]) * di_ref[...]
        acc = jnp.dot(h_ref[...], w_ref[0], preferred_element_type=jnp.float32)
        acc += jnp.dot(a1_ref[...], w_ref[1], preferred_element_type=jnp.float32)
        acc += jnp.dot(a2_ref[...], w_ref[2], preferred_element_type=jnp.float32)
        acc += jnp.dot(a3, w_ref[3], preferred_element_type=jnp.float32)
        y_ref[...] = acc + b_ref[...]

    return pl.pallas_call(
        body, out_shape=jax.ShapeDtypeStruct((N2, OUT_C), jnp.float32),
    )(h, a1, a2, p, dinv64, W2, b2)


def kernel(x, edge_index, W1, b1, W2, b2):
    src = edge_index[0]
    dst = edge_index[1]
    pad = E2 - EE
    pidx = lax.iota(jnp.int32, pad)
    src2 = jnp.concatenate([src, pidx % 128]).reshape(IDX_ROWS, 128)
    dst2 = jnp.concatenate([dst, NN + (pidx % 8)]).reshape(IDX_ROWS, 128)
    x2 = jnp.pad(x, ((0, N2 - NN), (0, 0)))
    zeros64 = jnp.zeros((N2, HID), jnp.float32)
    zeros16 = jnp.zeros((N2, 16), jnp.float32)
    ones16 = jnp.ones((128, 16), jnp.float32)

    z3 = _zmm3(x2, W1)
    p, degp = _app_first(z3, src2, dst2, zeros64, zeros16, ones16)
    z = _zmm012(x2, W1)
    dinv64 = _dinv(degp)
    p = _app_z(p, src2, dst2, zeros64, dinv64, z[2])
    p = _app_z(p, src2, dst2, zeros64, dinv64, z[1])
    p, h = _app_relu(p, src2, dst2, zeros64, dinv64, z[0], b1)
    p, a1 = _app_plain(p, src2, dst2, zeros64, dinv64)
    p, a2 = _app_plain(p, src2, dst2, zeros64, dinv64)
    y2 = _fin(h, a1, a2, p, dinv64, W2, b2)
    return y2[:NN]


# speedup vs baseline: 1.2030x; 1.1126x over previous
import functools

import jax
import jax.numpy as jnp
from jax import lax
from jax.experimental import pallas as pl
from jax.experimental.pallas import tpu as pltpu
from jax.experimental.pallas import tpu_sc as plsc

NN = 10000
EE = 320000
IN_C = 128
HID = 64
OUT_C = 128

NC = 2
NS = 16
NW = NC * NS

N2 = 10112
E2 = 327680
IDX_ROWS = E2 // 128
ROWS_PER_W = IDX_ROWS // NW
SUP = 8
NSUP = ROWS_PER_W // SUP
RPT = N2 // NS


def _sc_mesh():
    return plsc.VectorSubcoreMesh(core_axis_name="c", subcore_axis_name="s",
                                  num_cores=NC, num_subcores=NS)


def _sc_app_common(tbl, src2, dst2, z64, pout, acc, src_v, dst_v, rows_v,
                   gsem, ssem, isem, z16=None, ones16=None, dout=None,
                   dacc=None, ones_v=None):
    c = lax.axis_index("c")
    s = lax.axis_index("s")
    w = s * NC + c
    r0 = s * RPT
    pltpu.sync_copy(z64.at[pl.ds(r0, RPT)], acc.at[pl.ds(r0, RPT)])
    if dacc is not None:
        pltpu.sync_copy(z16.at[pl.ds(r0, RPT)], dacc.at[pl.ds(r0, RPT)])
        pltpu.sync_copy(ones16, ones_v)
    plsc.subcore_barrier()

    base = w * ROWS_PER_W

    def _scatter_waits(j):
        pltpu.make_async_copy(tbl.at[pl.ds(0, 128)],
                              rows_v.at[pl.ds(j * 128, 128)],
                              ssem.at[j]).wait()
        if dacc is not None:
            pltpu.make_async_copy(ones16, ones_v, ssem.at[j]).wait()

    def chunk(i, carry):
        ro = base + i * SUP
        par = lax.rem(i, 2)
        ic0 = pltpu.async_copy(src2.at[pl.ds(ro, SUP)], src_v.at[par], isem)
        ic1 = pltpu.async_copy(dst2.at[pl.ds(ro, SUP)], dst_v.at[par], isem)

        @pl.when(i > 0)
        def _():
            for j in range(SUP):
                _scatter_waits(j)
        ic0.wait()
        ic1.wait()

        gcs = [pltpu.async_copy(tbl.at[src_v.at[par, j]],
                                rows_v.at[pl.ds(j * 128, 128)],
                                gsem.at[j])
               for j in range(SUP)]
        for j in range(SUP):
            gcs[j].wait()
            pltpu.async_copy(rows_v.at[pl.ds(j * 128, 128)],
                             acc.at[dst_v.at[par, j]], ssem.at[j], add=True)
            if dacc is not None:
                pltpu.async_copy(ones_v, dacc.at[dst_v.at[par, j]],
                                 ssem.at[j], add=True)
        return carry

    lax.fori_loop(0, NSUP, chunk, 0)
    for j in range(SUP):
        _scatter_waits(j)
    plsc.subcore_barrier()
    pltpu.sync_copy(acc.at[pl.ds(r0, RPT)], pout.at[c, pl.ds(r0, RPT)])
    if dacc is not None:
        pltpu.sync_copy(dacc.at[pl.ds(r0, RPT)], dout.at[c, pl.ds(r0, RPT)])


@functools.partial(
    pl.kernel,
    out_type=(jax.ShapeDtypeStruct((NC, N2, HID), jnp.float32),
              jax.ShapeDtypeStruct((NC, N2, 16), jnp.float32)),
    mesh=_sc_mesh(),
    compiler_params=pltpu.CompilerParams(use_tc_tiling_on_sc=False),
    scratch_types=(
        pltpu.VMEM_SHARED((N2, HID), jnp.float32),
        pltpu.VMEM((2, SUP, 128), jnp.int32),
        pltpu.VMEM((2, SUP, 128), jnp.int32),
        pltpu.VMEM((SUP * 128, HID), jnp.float32),
        pltpu.SemaphoreType.DMA((SUP,)),
        pltpu.SemaphoreType.DMA((SUP,)),
        pltpu.SemaphoreType.DMA,
        pltpu.VMEM_SHARED((N2, 16), jnp.float32),
        pltpu.VMEM((128, 16), jnp.float32),
    ),
)
def _sc_app_deg(tbl, src2, dst2, z64, z16, ones16, pout, dout, acc, src_v,
                dst_v, rows_v, gsem, ssem, isem, dacc, ones_v):
    _sc_app_common(tbl, src2, dst2, z64, pout, acc, src_v, dst_v, rows_v,
                   gsem, ssem, isem, z16=z16, ones16=ones16, dout=dout,
                   dacc=dacc, ones_v=ones_v)


@functools.partial(
    pl.kernel,
    out_type=jax.ShapeDtypeStruct((NC, N2, HID), jnp.float32),
    mesh=_sc_mesh(),
    compiler_params=pltpu.CompilerParams(use_tc_tiling_on_sc=False),
    scratch_types=(
        pltpu.VMEM_SHARED((N2, HID), jnp.float32),
        pltpu.VMEM((2, SUP, 128), jnp.int32),
        pltpu.VMEM((2, SUP, 128), jnp.int32),
        pltpu.VMEM((SUP * 128, HID), jnp.float32),
        pltpu.SemaphoreType.DMA((SUP,)),
        pltpu.SemaphoreType.DMA((SUP,)),
        pltpu.SemaphoreType.DMA,
    ),
)
def _sc_app(tbl, src2, dst2, z64, pout, acc, src_v, dst_v, rows_v, gsem,
            ssem, isem):
    _sc_app_common(tbl, src2, dst2, z64, pout, acc, src_v, dst_v, rows_v,
                   gsem, ssem, isem)


GB = 8
BN = N2 // GB


def _zmm3(x2, W1):
    def body(x_ref, w_ref, z_ref):
        z_ref[...] = jnp.dot(x_ref[...], w_ref[3],
                             preferred_element_type=jnp.float32)

    return pl.pallas_call(
        body,
        grid=(GB,),
        in_specs=[pl.BlockSpec((BN, IN_C), lambda i: (i, 0)),
                  pl.BlockSpec((4, IN_C, HID), lambda i: (0, 0, 0))],
        out_specs=pl.BlockSpec((BN, HID), lambda i: (i, 0)),
        out_shape=jax.ShapeDtypeStruct((N2, HID), jnp.float32),
    )(x2, W1)


def _zmm012(x2, W1):
    def body(x_ref, w_ref, z_ref):
        for k in range(3):
            z_ref[k] = jnp.dot(x_ref[...], w_ref[k],
                               preferred_element_type=jnp.float32)

    return pl.pallas_call(
        body,
        grid=(GB,),
        in_specs=[pl.BlockSpec((BN, IN_C), lambda i: (i, 0)),
                  pl.BlockSpec((4, IN_C, HID), lambda i: (0, 0, 0))],
        out_specs=pl.BlockSpec((3, BN, HID), lambda i: (0, i, 0)),
        out_shape=jax.ShapeDtypeStruct((3, N2, HID), jnp.float32),
    )(x2, W1)


def _comb1(p, degp, z):
    def body(p_ref, dp_ref, z_ref, t_ref, di_ref):
        dinv = 1.0 / jnp.maximum(dp_ref[0] + dp_ref[1], 1.0)
        di_ref[...] = dinv
        t_ref[...] = (p_ref[0] + p_ref[1]) * dinv[:, :1] + z_ref[...]

    return pl.pallas_call(
        body,
        grid=(GB,),
        in_specs=[pl.BlockSpec((2, BN, HID), lambda i: (0, i, 0)),
                  pl.BlockSpec((2, BN, 16), lambda i: (0, i, 0)),
                  pl.BlockSpec((BN, HID), lambda i: (i, 0))],
        out_specs=(pl.BlockSpec((BN, HID), lambda i: (i, 0)),
                   pl.BlockSpec((BN, 16), lambda i: (i, 0))),
        out_shape=(jax.ShapeDtypeStruct((N2, HID), jnp.float32),
                   jax.ShapeDtypeStruct((N2, 16), jnp.float32)),
    )(p, degp, z)


def _comb_add(p, dinv, z):
    def body(p_ref, di_ref, z_ref, t_ref):
        t_ref[...] = (p_ref[0] + p_ref[1]) * di_ref[:, :1] + z_ref[...]

    return pl.pallas_call(
        body,
        grid=(GB,),
        in_specs=[pl.BlockSpec((2, BN, HID), lambda i: (0, i, 0)),
                  pl.BlockSpec((BN, 16), lambda i: (i, 0)),
                  pl.BlockSpec((BN, HID), lambda i: (i, 0))],
        out_specs=pl.BlockSpec((BN, HID), lambda i: (i, 0)),
        out_shape=jax.ShapeDtypeStruct((N2, HID), jnp.float32),
    )(p, dinv, z)


def _comb_relu(p, dinv, z, b1):
    def body(p_ref, di_ref, z_ref, b_ref, t_ref):
        t = (p_ref[0] + p_ref[1]) * di_ref[:, :1] + z_ref[...] + b_ref[...]
        t_ref[...] = jnp.maximum(t, 0.0)

    return pl.pallas_call(
        body,
        grid=(GB,),
        in_specs=[pl.BlockSpec((2, BN, HID), lambda i: (0, i, 0)),
                  pl.BlockSpec((BN, 16), lambda i: (i, 0)),
                  pl.BlockSpec((BN, HID), lambda i: (i, 0)),
                  pl.BlockSpec((HID,), lambda i: (0,))],
        out_specs=pl.BlockSpec((BN, HID), lambda i: (i, 0)),
        out_shape=jax.ShapeDtypeStruct((N2, HID), jnp.float32),
    )(p, dinv, z, b1)


def _comb_plain(p, dinv):
    def body(p_ref, di_ref, t_ref):
        t_ref[...] = (p_ref[0] + p_ref[1]) * di_ref[:, :1]

    return pl.pallas_call(
        body,
        grid=(GB,),
        in_specs=[pl.BlockSpec((2, BN, HID), lambda i: (0, i, 0)),
                  pl.BlockSpec((BN, 16), lambda i: (i, 0))],
        out_specs=pl.BlockSpec((BN, HID), lambda i: (i, 0)),
        out_shape=jax.ShapeDtypeStruct((N2, HID), jnp.float32),
    )(p, dinv)


def _fin(h, a1, a2, p, dinv, W2, b2):
    def body(h_ref, a1_ref, a2_ref, p_ref, di_ref, w_ref, b_ref, y_ref):
        a3 = (p_ref[0] + p_ref[1]) * di_ref[:, :1]
        acc = jnp.dot(h_ref[...], w_ref[0], preferred_element_type=jnp.float32)
        acc += jnp.dot(a1_ref[...], w_ref[1], preferred_element_type=jnp.float32)
        acc += jnp.dot(a2_ref[...], w_ref[2], preferred_element_type=jnp.float32)
        acc += jnp.dot(a3, w_ref[3], preferred_element_type=jnp.float32)
        y_ref[...] = acc + b_ref[...]

    return pl.pallas_call(
        body,
        grid=(GB,),
        in_specs=[pl.BlockSpec((BN, HID), lambda i: (i, 0)),
                  pl.BlockSpec((BN, HID), lambda i: (i, 0)),
                  pl.BlockSpec((BN, HID), lambda i: (i, 0)),
                  pl.BlockSpec((2, BN, HID), lambda i: (0, i, 0)),
                  pl.BlockSpec((BN, 16), lambda i: (i, 0)),
                  pl.BlockSpec((4, HID, OUT_C), lambda i: (0, 0, 0)),
                  pl.BlockSpec((OUT_C,), lambda i: (0,)),
                  ],
        out_specs=pl.BlockSpec((BN, OUT_C), lambda i: (i, 0)),
        out_shape=jax.ShapeDtypeStruct((N2, OUT_C), jnp.float32),
    )(h, a1, a2, p, dinv, W2, b2)


def kernel(x, edge_index, W1, b1, W2, b2):
    src = edge_index[0]
    dst = edge_index[1]
    pad = E2 - EE
    pidx = lax.iota(jnp.int32, pad)
    src2 = jnp.concatenate([src, pidx % 128]).reshape(IDX_ROWS, 128)
    dst2 = jnp.concatenate([dst, NN + (pidx % 8)]).reshape(IDX_ROWS, 128)
    x2 = jnp.pad(x, ((0, N2 - NN), (0, 0)))
    zeros64 = jnp.zeros((N2, HID), jnp.float32)
    zeros16 = jnp.zeros((N2, 16), jnp.float32)
    ones16 = jnp.ones((128, 16), jnp.float32)

    z3 = _zmm3(x2, W1)
    p, degp = _sc_app_deg(z3, src2, dst2, zeros64, zeros16, ones16)
    z = _zmm012(x2, W1)
    t, dinv = _comb1(p, degp, z[2])
    p = _sc_app(t, src2, dst2, zeros64)
    t = _comb_add(p, dinv, z[1])
    p = _sc_app(t, src2, dst2, zeros64)
    h = _comb_relu(p, dinv, z[0], b1)
    p = _sc_app(h, src2, dst2, zeros64)
    a1 = _comb_plain(p, dinv)
    p = _sc_app(a1, src2, dst2, zeros64)
    a2 = _comb_plain(p, dinv)
    p = _sc_app(a2, src2, dst2, zeros64)
    y2 = _fin(h, a1, a2, p, dinv, W2, b2)
    return y2[:NN]


# speedup vs baseline: 1.2126x; 1.0080x over previous
import functools

import jax
import jax.numpy as jnp
from jax import lax
from jax.experimental import pallas as pl
from jax.experimental.pallas import tpu as pltpu
from jax.experimental.pallas import tpu_sc as plsc

NN = 10000
EE = 320000
IN_C = 128
HID = 64
OUT_C = 128

NC = 2
NS = 16
NW = NC * NS

N2 = 10112
E2 = 327680
IDX_ROWS = E2 // 128
ROWS_PER_W = IDX_ROWS // NW
SUP = 8
NSUP = ROWS_PER_W // SUP
RPT = N2 // NS


def _sc_mesh():
    return plsc.VectorSubcoreMesh(core_axis_name="c", subcore_axis_name="s",
                                  num_cores=NC, num_subcores=NS)


def _sc_app_common(tbl, src2, dst2, z64, pout, acc, src_v, dst_v, rows_v,
                   gsem, ssem, isem, z16=None, ones16=None, dout=None,
                   dacc=None, ones_v=None):
    c = lax.axis_index("c")
    s = lax.axis_index("s")
    w = s * NC + c
    r0 = s * RPT
    pltpu.sync_copy(z64.at[pl.ds(r0, RPT)], acc.at[pl.ds(r0, RPT)])
    if dacc is not None:
        pltpu.sync_copy(z16.at[pl.ds(r0, RPT)], dacc.at[pl.ds(r0, RPT)])
        pltpu.sync_copy(ones16, ones_v)
    plsc.subcore_barrier()

    base = w * ROWS_PER_W

    def _scatter_waits(j):
        pltpu.make_async_copy(tbl.at[pl.ds(0, 128)],
                              rows_v.at[pl.ds(j * 128, 128)],
                              ssem.at[j]).wait()
        if dacc is not None:
            pltpu.make_async_copy(ones16, ones_v, ssem.at[j]).wait()

    def chunk(i, carry):
        ro = base + i * SUP
        par = lax.rem(i, 2)
        ic0 = pltpu.async_copy(src2.at[pl.ds(ro, SUP)], src_v.at[par], isem)
        ic1 = pltpu.async_copy(dst2.at[pl.ds(ro, SUP)], dst_v.at[par], isem)

        @pl.when(i > 0)
        def _():
            for j in range(SUP):
                _scatter_waits(j)
        ic0.wait()
        ic1.wait()

        gcs = [pltpu.async_copy(tbl.at[src_v.at[par, j]],
                                rows_v.at[pl.ds(j * 128, 128)],
                                gsem.at[j])
               for j in range(SUP)]
        for j in range(SUP):
            gcs[j].wait()
            pltpu.async_copy(rows_v.at[pl.ds(j * 128, 128)],
                             acc.at[dst_v.at[par, j]], ssem.at[j], add=True)
            if dacc is not None:
                pltpu.async_copy(ones_v, dacc.at[dst_v.at[par, j]],
                                 ssem.at[j], add=True)
        return carry

    lax.fori_loop(0, NSUP, chunk, 0)
    for j in range(SUP):
        _scatter_waits(j)
    plsc.subcore_barrier()
    pltpu.sync_copy(acc.at[pl.ds(r0, RPT)], pout.at[c, pl.ds(r0, RPT)])
    if dacc is not None:
        pltpu.sync_copy(dacc.at[pl.ds(r0, RPT)], dout.at[c, pl.ds(r0, RPT)])


@functools.partial(
    pl.kernel,
    out_type=(jax.ShapeDtypeStruct((NC, N2, HID), jnp.float32),
              jax.ShapeDtypeStruct((NC, N2, 16), jnp.float32)),
    mesh=_sc_mesh(),
    compiler_params=pltpu.CompilerParams(use_tc_tiling_on_sc=False),
    scratch_types=(
        pltpu.VMEM_SHARED((N2, HID), jnp.float32),
        pltpu.VMEM((2, SUP, 128), jnp.int32),
        pltpu.VMEM((2, SUP, 128), jnp.int32),
        pltpu.VMEM((SUP * 128, HID), jnp.float32),
        pltpu.SemaphoreType.DMA((SUP,)),
        pltpu.SemaphoreType.DMA((SUP,)),
        pltpu.SemaphoreType.DMA,
        pltpu.VMEM_SHARED((N2, 16), jnp.float32),
        pltpu.VMEM((128, 16), jnp.float32),
    ),
)
def _sc_app_deg(tbl, src2, dst2, z64, z16, ones16, pout, dout, acc, src_v,
                dst_v, rows_v, gsem, ssem, isem, dacc, ones_v):
    _sc_app_common(tbl, src2, dst2, z64, pout, acc, src_v, dst_v, rows_v,
                   gsem, ssem, isem, z16=z16, ones16=ones16, dout=dout,
                   dacc=dacc, ones_v=ones_v)


@functools.partial(
    pl.kernel,
    out_type=jax.ShapeDtypeStruct((NC, N2, HID), jnp.float32),
    mesh=_sc_mesh(),
    compiler_params=pltpu.CompilerParams(use_tc_tiling_on_sc=False),
    scratch_types=(
        pltpu.VMEM_SHARED((N2, HID), jnp.float32),
        pltpu.VMEM((2, SUP, 128), jnp.int32),
        pltpu.VMEM((2, SUP, 128), jnp.int32),
        pltpu.VMEM((SUP * 128, HID), jnp.float32),
        pltpu.SemaphoreType.DMA((SUP,)),
        pltpu.SemaphoreType.DMA((SUP,)),
        pltpu.SemaphoreType.DMA,
    ),
)
def _sc_app(tbl, src2, dst2, z64, pout, acc, src_v, dst_v, rows_v, gsem,
            ssem, isem):
    _sc_app_common(tbl, src2, dst2, z64, pout, acc, src_v, dst_v, rows_v,
                   gsem, ssem, isem)


GB = 8
BN = N2 // GB


def _zmm3(x2, W1):
    def body(x_ref, w_ref, z_ref):
        z_ref[...] = jnp.dot(x_ref[...], w_ref[3],
                             preferred_element_type=jnp.float32)

    return pl.pallas_call(
        body,
        grid=(GB,),
        in_specs=[pl.BlockSpec((BN, IN_C), lambda i: (i, 0)),
                  pl.BlockSpec((4, IN_C, HID), lambda i: (0, 0, 0))],
        out_specs=pl.BlockSpec((BN, HID), lambda i: (i, 0)),
        out_shape=jax.ShapeDtypeStruct((N2, HID), jnp.float32),
    )(x2, W1)


def _zmm_k(x2, W1, k):
    def body(x_ref, w_ref, z_ref):
        z_ref[...] = jnp.dot(x_ref[...], w_ref[k],
                             preferred_element_type=jnp.float32)

    return pl.pallas_call(
        body,
        grid=(GB,),
        in_specs=[pl.BlockSpec((BN, IN_C), lambda i: (i, 0)),
                  pl.BlockSpec((4, IN_C, HID), lambda i: (0, 0, 0))],
        out_specs=pl.BlockSpec((BN, HID), lambda i: (i, 0)),
        out_shape=jax.ShapeDtypeStruct((N2, HID), jnp.float32),
    )(x2, W1)


def _comb1(p, degp, z):
    def body(p_ref, dp_ref, z_ref, t_ref, di_ref):
        dinv = 1.0 / jnp.maximum(dp_ref[0] + dp_ref[1], 1.0)
        di_ref[...] = dinv
        t_ref[...] = (p_ref[0] + p_ref[1]) * dinv[:, :1] + z_ref[...]

    return pl.pallas_call(
        body,
        grid=(GB,),
        in_specs=[pl.BlockSpec((2, BN, HID), lambda i: (0, i, 0)),
                  pl.BlockSpec((2, BN, 16), lambda i: (0, i, 0)),
                  pl.BlockSpec((BN, HID), lambda i: (i, 0))],
        out_specs=(pl.BlockSpec((BN, HID), lambda i: (i, 0)),
                   pl.BlockSpec((BN, 16), lambda i: (i, 0))),
        out_shape=(jax.ShapeDtypeStruct((N2, HID), jnp.float32),
                   jax.ShapeDtypeStruct((N2, 16), jnp.float32)),
    )(p, degp, z)


def _comb_add(p, dinv, z):
    def body(p_ref, di_ref, z_ref, t_ref):
        t_ref[...] = (p_ref[0] + p_ref[1]) * di_ref[:, :1] + z_ref[...]

    return pl.pallas_call(
        body,
        grid=(GB,),
        in_specs=[pl.BlockSpec((2, BN, HID), lambda i: (0, i, 0)),
                  pl.BlockSpec((BN, 16), lambda i: (i, 0)),
                  pl.BlockSpec((BN, HID), lambda i: (i, 0))],
        out_specs=pl.BlockSpec((BN, HID), lambda i: (i, 0)),
        out_shape=jax.ShapeDtypeStruct((N2, HID), jnp.float32),
    )(p, dinv, z)


def _comb_relu(p, dinv, z, b1):
    def body(p_ref, di_ref, z_ref, b_ref, t_ref):
        t = (p_ref[0] + p_ref[1]) * di_ref[:, :1] + z_ref[...] + b_ref[...]
        t_ref[...] = jnp.maximum(t, 0.0)

    return pl.pallas_call(
        body,
        grid=(GB,),
        in_specs=[pl.BlockSpec((2, BN, HID), lambda i: (0, i, 0)),
                  pl.BlockSpec((BN, 16), lambda i: (i, 0)),
                  pl.BlockSpec((BN, HID), lambda i: (i, 0)),
                  pl.BlockSpec((HID,), lambda i: (0,))],
        out_specs=pl.BlockSpec((BN, HID), lambda i: (i, 0)),
        out_shape=jax.ShapeDtypeStruct((N2, HID), jnp.float32),
    )(p, dinv, z, b1)


def _comb_plain(p, dinv):
    def body(p_ref, di_ref, t_ref):
        t_ref[...] = (p_ref[0] + p_ref[1]) * di_ref[:, :1]

    return pl.pallas_call(
        body,
        grid=(GB,),
        in_specs=[pl.BlockSpec((2, BN, HID), lambda i: (0, i, 0)),
                  pl.BlockSpec((BN, 16), lambda i: (i, 0))],
        out_specs=pl.BlockSpec((BN, HID), lambda i: (i, 0)),
        out_shape=jax.ShapeDtypeStruct((N2, HID), jnp.float32),
    )(p, dinv)


def _fin(h, a1, a2, p, dinv, W2, b2):
    def body(h_ref, a1_ref, a2_ref, p_ref, di_ref, w_ref, b_ref, y_ref):
        a3 = (p_ref[0] + p_ref[1]) * di_ref[:, :1]
        acc = jnp.dot(h_ref[...], w_ref[0], preferred_element_type=jnp.float32)
        acc += jnp.dot(a1_ref[...], w_ref[1], preferred_element_type=jnp.float32)
        acc += jnp.dot(a2_ref[...], w_ref[2], preferred_element_type=jnp.float32)
        acc += jnp.dot(a3, w_ref[3], preferred_element_type=jnp.float32)
        y_ref[...] = acc + b_ref[...]

    return pl.pallas_call(
        body,
        grid=(GB,),
        in_specs=[pl.BlockSpec((BN, HID), lambda i: (i, 0)),
                  pl.BlockSpec((BN, HID), lambda i: (i, 0)),
                  pl.BlockSpec((BN, HID), lambda i: (i, 0)),
                  pl.BlockSpec((2, BN, HID), lambda i: (0, i, 0)),
                  pl.BlockSpec((BN, 16), lambda i: (i, 0)),
                  pl.BlockSpec((4, HID, OUT_C), lambda i: (0, 0, 0)),
                  pl.BlockSpec((OUT_C,), lambda i: (0,)),
                  ],
        out_specs=pl.BlockSpec((BN, OUT_C), lambda i: (i, 0)),
        out_shape=jax.ShapeDtypeStruct((N2, OUT_C), jnp.float32),
    )(h, a1, a2, p, dinv, W2, b2)


def kernel(x, edge_index, W1, b1, W2, b2):
    src = edge_index[0]
    dst = edge_index[1]
    pad = E2 - EE
    pidx = lax.iota(jnp.int32, pad)
    src2 = jnp.concatenate([src, pidx % 128]).reshape(IDX_ROWS, 128)
    dst2 = jnp.concatenate([dst, NN + (pidx % 8)]).reshape(IDX_ROWS, 128)
    x2 = jnp.pad(x, ((0, N2 - NN), (0, 0)))
    zeros64 = jnp.zeros((N2, HID), jnp.float32)
    zeros16 = jnp.zeros((N2, 16), jnp.float32)
    ones16 = jnp.ones((128, 16), jnp.float32)

    z3 = _zmm3(x2, W1)
    p, degp = _sc_app_deg(z3, src2, dst2, zeros64, zeros16, ones16)
    z2 = _zmm_k(x2, W1, 2)
    t, dinv = _comb1(p, degp, z2)
    p = _sc_app(t, src2, dst2, zeros64)
    z1 = _zmm_k(x2, W1, 1)
    t = _comb_add(p, dinv, z1)
    p = _sc_app(t, src2, dst2, zeros64)
    z0 = _zmm_k(x2, W1, 0)
    h = _comb_relu(p, dinv, z0, b1)
    p = _sc_app(h, src2, dst2, zeros64)
    a1 = _comb_plain(p, dinv)
    p = _sc_app(a1, src2, dst2, zeros64)
    a2 = _comb_plain(p, dinv)
    p = _sc_app(a2, src2, dst2, zeros64)
    y2 = _fin(h, a1, a2, p, dinv, W2, b2)
    return y2[:NN]
